# Initial kernel scaffold; baseline (speedup 1.0000x reference)
import jax
import jax.numpy as jnp
from jax.experimental import pallas as pl


def kernel(x_i, edge_index_i, edge_attr_i, batch_i, x_j, edge_index_j, edge_attr_j, batch_j, Wq, bq, Wk, bk, Wv, bv, We, be, Wskip, bskip, ln_gamma, ln_beta, W_emb, b_emb):
    raise NotImplementedError("write your pallas kernel here")


# speedup vs baseline: 21.9872x; 21.9872x over previous
import functools

import jax
import jax.numpy as jnp
from jax import lax
from jax.experimental import pallas as pl
from jax.experimental.pallas import tpu as pltpu
from jax.experimental.pallas import tpu_sc as plsc

N = 10000
E = 320000
D = 128
H = 4
HD = D // H
L = 4
G = 64
DE = 16
DH = D // 2
AW = DH + 16

NC, NS = 2, 16
EPT = E // NS
B = 80
NBLK = EPT // B
RPT = N // NS
ZR = 125

_LANE = 16


def _sc_edge(qh, kvh, keh, lyr, eidx):
    mesh = plsc.VectorSubcoreMesh(core_axis_name="c", subcore_axis_name="s")

    @functools.partial(
        pl.kernel,
        out_type=jax.ShapeDtypeStruct((NC, N, AW), jnp.float32),
        mesh=mesh,
        compiler_params=pltpu.CompilerParams(use_tc_tiling_on_sc=False,
                                             needs_layout_passes=False),
        scratch_types=[
            pltpu.VMEM((2, B), jnp.int32),
            pltpu.VMEM((B, DH), jnp.float32),
            pltpu.VMEM((B, 2 * DH), jnp.float32),
            pltpu.VMEM((B, DH), jnp.float32),
            pltpu.VMEM((B, AW), jnp.float32),
            pltpu.VMEM((ZR, AW), jnp.float32),
            pltpu.VMEM_SHARED((N, AW), jnp.float32),
            pltpu.SemaphoreType.DMA,
            pltpu.SemaphoreType.DMA,
            pltpu.SemaphoreType.DMA,
        ],
    )
    def k(qh_hbm, kvh_hbm, keh_hbm, eidx_hbm, out_hbm,
          idxb, qb, kvb, keb, ob, zb, agg, sq, skv, ske):
        cid = lax.axis_index("c")
        sid = lax.axis_index("s")

        @pl.loop(0, ZR)
        def _(r):
            for c in range(AW // _LANE):
                zb[r, pl.ds(c * _LANE, _LANE)] = jnp.zeros((_LANE,), jnp.float32)

        row0 = sid * RPT
        for z in range(RPT // ZR):
            pltpu.sync_copy(zb, agg.at[pl.ds(row0 + z * ZR, ZR)])
        plsc.subcore_barrier()

        lane = lax.iota(jnp.int32, _LANE)
        kslab = cid * L + lyr

        @pl.loop(0, NBLK)
        def _(g):
            base = sid * EPT + g * B
            pltpu.sync_copy(eidx_hbm.at[:, pl.ds(base, B)], idxb)
            cq = pltpu.async_copy(qh_hbm.at[cid].at[idxb.at[1]], qb, sq)
            ck = pltpu.async_copy(kvh_hbm.at[cid].at[idxb.at[0]], kvb, skv)
            ce = pltpu.async_copy(keh_hbm.at[kslab].at[pl.ds(base, B)], keb, ske)
            cq.wait()
            ck.wait()
            ce.wait()

            @pl.loop(0, B)
            def _(e):
                sacc = jnp.zeros((_LANE,), jnp.float32)
                for h in range(H // NC):
                    c0 = HD * h
                    c1 = c0 + _LANE
                    ke0 = keb[e, pl.ds(c0, _LANE)]
                    ke1 = keb[e, pl.ds(c1, _LANE)]
                    q0 = qb[e, pl.ds(c0, _LANE)]
                    q1 = qb[e, pl.ds(c1, _LANE)]
                    k0 = kvb[e, pl.ds(c0, _LANE)] + ke0
                    k1 = kvb[e, pl.ds(c1, _LANE)] + ke1
                    t = q0 * k0 + q1 * k1
                    lg = jnp.sum(t)
                    a = jnp.exp(jnp.full((_LANE,), lg, jnp.float32))
                    v0 = kvb[e, pl.ds(DH + c0, _LANE)] + ke0
                    v1 = kvb[e, pl.ds(DH + c1, _LANE)] + ke1
                    ob[e, pl.ds(c0, _LANE)] = a * v0
                    ob[e, pl.ds(c1, _LANE)] = a * v1
                    sacc = jnp.where(lane == h, a, sacc)
                ob[e, pl.ds(DH, _LANE)] = sacc

            pltpu.sync_copy(ob, agg.at[idxb.at[1]], add=True)

        plsc.subcore_barrier()

        for z in range(RPT // ZR):
            r = row0 + z * ZR
            pltpu.sync_copy(agg.at[pl.ds(r, ZR)],
                            out_hbm.at[cid].at[pl.ds(r, ZR)])

    return k(qh, kvh, keh, eidx)


NB = 2000
EB = 8000


def _ke_body(ea_ref, we_ref, be_ref, out_ref):
    ke = jnp.dot(ea_ref[...], we_ref[0],
                 preferred_element_type=jnp.float32) + be_ref[0]
    out_ref[0, 0] = ke[:, :DH]
    out_ref[1, 0] = ke[:, DH:]


def _tc_ke(eattr, We, be):
    out = pl.pallas_call(
        _ke_body,
        grid=(L, E // EB),
        in_specs=[
            pl.BlockSpec((EB, DE), lambda l, i: (i, 0)),
            pl.BlockSpec((1, DE, D), lambda l, i: (l, 0, 0)),
            pl.BlockSpec((1, 1, D), lambda l, i: (l, 0, 0)),
        ],
        out_specs=pl.BlockSpec((NC, 1, EB, DH), lambda l, i: (0, l, i, 0)),
        out_shape=jax.ShapeDtypeStruct((NC, L, E, DH), jnp.float32),
    )(eattr, We, be.reshape(L, 1, D))
    return out.reshape(NC * L, E, DH)


_INV_SQRT_HD = 1.0 / (HD ** 0.5)


def _split_qkv(qf, kf, vf, q_ref, kv_ref):
    q_ref[0] = qf[:, :DH]
    q_ref[1] = qf[:, DH:]
    kv_ref[0, :, :DH] = kf[:, :DH]
    kv_ref[0, :, DH:] = vf[:, :DH]
    kv_ref[1, :, :DH] = kf[:, DH:]
    kv_ref[1, :, DH:] = vf[:, DH:]


def _qkv_body(h_ref, wq_ref, bq_ref, wk_ref, bk_ref, wv_ref, bv_ref,
              q_ref, kv_ref):
    h = h_ref[...]
    qf = (jnp.dot(h, wq_ref[...], preferred_element_type=jnp.float32)
          + bq_ref[...]) * _INV_SQRT_HD
    kf = jnp.dot(h, wk_ref[...], preferred_element_type=jnp.float32) + bk_ref[...]
    vf = jnp.dot(h, wv_ref[...], preferred_element_type=jnp.float32) + bv_ref[...]
    _split_qkv(qf, kf, vf, q_ref, kv_ref)


def _tc_qkv0(x, wq, bq, wk, bk, wv, bv):
    wspec = pl.BlockSpec((D, D), lambda i: (0, 0))
    bspec = pl.BlockSpec((1, D), lambda i: (0, 0))
    return pl.pallas_call(
        _qkv_body,
        grid=(N // NB,),
        in_specs=[pl.BlockSpec((NB, D), lambda i: (i, 0)),
                  wspec, bspec, wspec, bspec, wspec, bspec],
        out_specs=[pl.BlockSpec((NC, NB, DH), lambda i: (0, i, 0)),
                   pl.BlockSpec((NC, NB, 2 * DH), lambda i: (0, i, 0))],
        out_shape=[jax.ShapeDtypeStruct((NC, N, DH), jnp.float32),
                   jax.ShapeDtypeStruct((NC, N, 2 * DH), jnp.float32)],
    )(x, wq.reshape(D, D), bq.reshape(1, D), wk.reshape(D, D),
      bk.reshape(1, D), wv.reshape(D, D), bv.reshape(1, D))


def _agg_norm(parts):
    agg = jnp.concatenate([parts[0, :, :DH], parts[1, :, :DH]], axis=1)
    s = jnp.concatenate([parts[0, :, DH:DH + 2],
                         parts[1, :, DH:DH + 2]], axis=1) + 1e-16
    rr = lax.broadcasted_iota(jnp.int32, (H, D), 0)
    cc = lax.broadcasted_iota(jnp.int32, (H, D), 1)
    sel = jnp.where(cc // HD == rr, 1.0, 0.0).astype(jnp.float32)
    dv = jnp.dot(s, sel, preferred_element_type=jnp.float32)
    return agg / dv


def _upd_body(p_ref, h_ref, ws_ref, bs_ref, wq_ref, bq_ref, wk_ref, bk_ref,
              wv_ref, bv_ref, h_out, q_ref, kv_ref):
    res = _agg_norm(p_ref[...])
    hn = res + jnp.dot(h_ref[...], ws_ref[...],
                       preferred_element_type=jnp.float32) + bs_ref[...]
    hn = jnp.maximum(hn, 0.0)
    h_out[...] = hn
    qf = (jnp.dot(hn, wq_ref[...], preferred_element_type=jnp.float32)
          + bq_ref[...]) * _INV_SQRT_HD
    kf = jnp.dot(hn, wk_ref[...], preferred_element_type=jnp.float32) + bk_ref[...]
    vf = jnp.dot(hn, wv_ref[...], preferred_element_type=jnp.float32) + bv_ref[...]
    _split_qkv(qf, kf, vf, q_ref, kv_ref)


def _tc_update(parts, h, ws, bs, wq, bq, wk, bk, wv, bv):
    wspec = pl.BlockSpec((D, D), lambda i: (0, 0))
    bspec = pl.BlockSpec((1, D), lambda i: (0, 0))
    return pl.pallas_call(
        _upd_body,
        grid=(N // NB,),
        in_specs=[pl.BlockSpec((NC, NB, AW), lambda i: (0, i, 0)),
                  pl.BlockSpec((NB, D), lambda i: (i, 0)),
                  wspec, bspec, wspec, bspec, wspec, bspec, wspec, bspec],
        out_specs=[pl.BlockSpec((NB, D), lambda i: (i, 0)),
                   pl.BlockSpec((NC, NB, DH), lambda i: (0, i, 0)),
                   pl.BlockSpec((NC, NB, 2 * DH), lambda i: (0, i, 0))],
        out_shape=[jax.ShapeDtypeStruct((N, D), jnp.float32),
                   jax.ShapeDtypeStruct((NC, N, DH), jnp.float32),
                   jax.ShapeDtypeStruct((NC, N, 2 * DH), jnp.float32)],
    )(parts, h, ws.reshape(D, D), bs.reshape(1, D), wq.reshape(D, D),
      bq.reshape(1, D), wk.reshape(D, D), bk.reshape(1, D),
      wv.reshape(D, D), bv.reshape(1, D))


def _last_body(p_ref, h_ref, ws_ref, bs_ref, h_out):
    res = _agg_norm(p_ref[...])
    h_out[...] = res + jnp.dot(h_ref[...], ws_ref[...],
                               preferred_element_type=jnp.float32) + bs_ref[...]


def _tc_last(parts, h, ws, bs):
    return pl.pallas_call(
        _last_body,
        grid=(N // NB,),
        in_specs=[pl.BlockSpec((NC, NB, AW), lambda i: (0, i, 0)),
                  pl.BlockSpec((NB, D), lambda i: (i, 0)),
                  pl.BlockSpec((D, D), lambda i: (0, 0)),
                  pl.BlockSpec((1, D), lambda i: (0, 0))],
        out_specs=pl.BlockSpec((NB, D), lambda i: (i, 0)),
        out_shape=jax.ShapeDtypeStruct((N, D), jnp.float32),
    )(parts, h, ws.reshape(D, D), bs.reshape(1, D))


NP = 10240
NB2 = 2048


def _embed(pooled, gamma, beta, wemb, bemb):
    mu = jnp.mean(pooled, axis=-1, keepdims=True)
    var = jnp.mean((pooled - mu) ** 2, axis=-1, keepdims=True)
    z = (pooled - mu) * lax.rsqrt(var + 1e-5) * gamma + beta
    return jnp.maximum(
        jnp.dot(z, wemb, preferred_element_type=jnp.float32) + bemb, 0.0)


def _final_body(hi_ref, bi_ref, hj_ref, bj_ref, g_ref, b_ref, we_ref, be_ref,
                out_ref, pi_acc, pj_acc):
    i = pl.program_id(0)

    @pl.when(i == 0)
    def _():
        pi_acc[...] = jnp.zeros((G, D), jnp.float32)
        pj_acc[...] = jnp.zeros((G, D), jnp.float32)

    rows = lax.broadcasted_iota(jnp.int32, (G, NB2), 0)
    mi = jnp.where(rows == bi_ref[0, 0][None, :], 1.0, 0.0).astype(jnp.float32)
    mj = jnp.where(rows == bj_ref[0, 0][None, :], 1.0, 0.0).astype(jnp.float32)
    pi_acc[...] += jnp.dot(mi, hi_ref[...], preferred_element_type=jnp.float32)
    pj_acc[...] += jnp.dot(mj, hj_ref[...], preferred_element_type=jnp.float32)

    @pl.when(i == NP // NB2 - 1)
    def _():
        ei = _embed(pi_acc[...], g_ref[...], b_ref[...], we_ref[...], be_ref[...])
        ej = _embed(pj_acc[...], g_ref[...], b_ref[...], we_ref[...], be_ref[...])
        num = jnp.sum(ei * ej, axis=-1, keepdims=True)
        na = jnp.maximum(jnp.sqrt(jnp.sum(ei * ei, axis=-1, keepdims=True)), 1e-8)
        nb = jnp.maximum(jnp.sqrt(jnp.sum(ej * ej, axis=-1, keepdims=True)), 1e-8)
        out_ref[...] = jnp.broadcast_to(num / (na * nb), (G, D))


def _tc_final(hi, bi, hj, bj, gamma, beta, wemb, bemb):
    pad = ((0, NP - N), (0, 0))
    hi = jnp.pad(hi, pad)
    hj = jnp.pad(hj, pad)
    bir = jnp.pad(bi, (0, NP - N), constant_values=G).reshape(NP // NB2, 1, NB2)
    bjr = jnp.pad(bj, (0, NP - N), constant_values=G).reshape(NP // NB2, 1, NB2)
    hspec = pl.BlockSpec((NB2, D), lambda i: (i, 0))
    ispec = pl.BlockSpec((1, 1, NB2), lambda i: (i, 0, 0))
    pspec = pl.BlockSpec((1, D), lambda i: (0, 0))
    wspec = pl.BlockSpec((D, D), lambda i: (0, 0))
    out = pl.pallas_call(
        _final_body,
        grid=(NP // NB2,),
        in_specs=[hspec, ispec, hspec, ispec, pspec, pspec, wspec, pspec],
        out_specs=pl.BlockSpec((G, D), lambda i: (0, 0)),
        out_shape=jax.ShapeDtypeStruct((G, D), jnp.float32),
        scratch_shapes=[pltpu.VMEM((G, D), jnp.float32),
                        pltpu.VMEM((G, D), jnp.float32)],
    )(hi, bir, hj, bjr, gamma.reshape(1, D), beta.reshape(1, D),
      wemb, bemb.reshape(1, D))
    return out[:, 0]


def kernel(x_i, edge_index_i, edge_attr_i, batch_i,
           x_j, edge_index_j, edge_attr_j, batch_j,
           Wq, bq, Wk, bk, Wv, bv, We, be, Wskip, bskip,
           ln_gamma, ln_beta, W_emb, b_emb):
    ke_i = _tc_ke(edge_attr_i, We, be)
    ke_j = _tc_ke(edge_attr_j, We, be)

    hi = x_i
    hj = x_j
    qi, kvi = _tc_qkv0(x_i, Wq[0], bq[0], Wk[0], bk[0], Wv[0], bv[0])
    qj, kvj = _tc_qkv0(x_j, Wq[0], bq[0], Wk[0], bk[0], Wv[0], bv[0])

    for l in range(L):
        pi = _sc_edge(qi, kvi, ke_i, l, edge_index_i)
        pj = _sc_edge(qj, kvj, ke_j, l, edge_index_j)
        if l < L - 1:
            hi, qi, kvi = _tc_update(pi, hi, Wskip[l], bskip[l], Wq[l + 1],
                                     bq[l + 1], Wk[l + 1], bk[l + 1],
                                     Wv[l + 1], bv[l + 1])
            hj, qj, kvj = _tc_update(pj, hj, Wskip[l], bskip[l], Wq[l + 1],
                                     bq[l + 1], Wk[l + 1], bk[l + 1],
                                     Wv[l + 1], bv[l + 1])
        else:
            hi = _tc_last(pi, hi, Wskip[l], bskip[l])
            hj = _tc_last(pj, hj, Wskip[l], bskip[l])

    return _tc_final(hi, batch_i, hj, batch_j,
                     ln_gamma, ln_beta, W_emb, b_emb)


# speedup vs baseline: 29.1854x; 1.3274x over previous
import functools

import jax
import jax.numpy as jnp
from jax import lax
from jax.experimental import pallas as pl
from jax.experimental.pallas import tpu as pltpu
from jax.experimental.pallas import tpu_sc as plsc

N = 10000
E = 320000
D = 128
H = 4
HD = D // H
L = 4
G = 64
DE = 16
DH = D // 2
AW = DH + 16

NC, NS = 2, 16
EPT = E // NS
B = 80
NBLK = EPT // B
RPT = N // NS
ZR = 125

_LANE = 16


def _sc_edge(qh, kvh, keh, lyr, eidx3):
    mesh = plsc.VectorSubcoreMesh(core_axis_name="c", subcore_axis_name="s")
    RB = NBLK

    @functools.partial(
        pl.kernel,
        out_type=jax.ShapeDtypeStruct((NC, N, AW), jnp.float32),
        mesh=mesh,
        compiler_params=pltpu.CompilerParams(use_tc_tiling_on_sc=False,
                                             needs_layout_passes=False),
        scratch_types=[
            pltpu.VMEM((2, B), jnp.int32),
            pltpu.VMEM((2, B), jnp.int32),
            pltpu.VMEM((B,), jnp.int32),
            pltpu.VMEM((B,), jnp.int32),
            pltpu.VMEM((B, DH), jnp.float32),
            pltpu.VMEM((B, DH), jnp.float32),
            pltpu.VMEM((B, 2 * DH), jnp.float32),
            pltpu.VMEM((B, 2 * DH), jnp.float32),
            pltpu.VMEM((B, DH), jnp.float32),
            pltpu.VMEM((B, DH), jnp.float32),
            pltpu.VMEM((B, AW), jnp.float32),
            pltpu.VMEM((B, AW), jnp.float32),
            pltpu.VMEM((ZR, AW), jnp.float32),
            pltpu.VMEM_SHARED((N, AW), jnp.float32),
            pltpu.SemaphoreType.DMA,
            pltpu.SemaphoreType.DMA,
            pltpu.SemaphoreType.DMA,
            pltpu.SemaphoreType.DMA,
            pltpu.SemaphoreType.DMA,
            pltpu.SemaphoreType.DMA,
            pltpu.SemaphoreType.DMA,
            pltpu.SemaphoreType.DMA,
            pltpu.SemaphoreType.DMA,
            pltpu.SemaphoreType.DMA,
        ],
    )
    def k(qh_hbm, kvh_hbm, keh_hbm, eidx_hbm, out_hbm,
          ix0, ix1, sd0, sd1, qb0, qb1, kvb0, kvb1, keb0, keb1, ob0, ob1,
          zb, agg, si0, si1, sq0, sq1, skv0, skv1, ske0, ske1, ss0, ss1):
        cid = lax.axis_index("c")
        sid = lax.axis_index("s")
        ixs, sds = (ix0, ix1), (sd0, sd1)
        qbs, kvbs, kebs, obs = (qb0, qb1), (kvb0, kvb1), (keb0, keb1), (ob0, ob1)
        sis, sqs = (si0, si1), (sq0, sq1)
        skvs, skes, sss = (skv0, skv1), (ske0, ske1), (ss0, ss1)

        @pl.loop(0, ZR)
        def _(r):
            for c in range(AW // _LANE):
                zb[r, pl.ds(c * _LANE, _LANE)] = jnp.zeros((_LANE,), jnp.float32)

        row0 = sid * RPT
        for z in range(RPT // ZR):
            pltpu.sync_copy(zb, agg.at[pl.ds(row0 + z * ZR, ZR)])
        plsc.subcore_barrier()

        lane = lax.iota(jnp.int32, _LANE)
        kslab = cid * L + lyr

        def idx_issue(b, g):
            pltpu.async_copy(eidx_hbm.at[sid * RB + g], ixs[b], sis[b])

        def idx_wait(b):
            pltpu.make_async_copy(eidx_hbm.at[0], ixs[b], sis[b]).wait()

        def gather_issue(b, g):
            pltpu.async_copy(qh_hbm.at[cid].at[ixs[b].at[1]], qbs[b], sqs[b])
            pltpu.async_copy(kvh_hbm.at[cid].at[ixs[b].at[0]], kvbs[b], skvs[b])
            pltpu.async_copy(keh_hbm.at[kslab].at[pl.ds((sid * RB + g) * B, B)],
                             kebs[b], skes[b])

        def gather_wait(b):
            pltpu.make_async_copy(qh_hbm.at[cid].at[ixs[b].at[1]],
                                  qbs[b], sqs[b]).wait()
            pltpu.make_async_copy(kvh_hbm.at[cid].at[ixs[b].at[0]],
                                  kvbs[b], skvs[b]).wait()
            pltpu.make_async_copy(keh_hbm.at[kslab].at[pl.ds(0, B)],
                                  kebs[b], skes[b]).wait()

        def dst_save(b):
            for c in range(B // _LANE):
                sds[b][pl.ds(c * _LANE, _LANE)] = ixs[b][1, pl.ds(c * _LANE, _LANE)]

        def scatter_issue(b):
            pltpu.async_copy(obs[b], agg.at[sds[b]], sss[b], add=True)

        def scatter_wait(b):
            pltpu.make_async_copy(obs[b], agg.at[sds[b]], sss[b]).wait()

        def compute(b):
            qb, kvb, keb, ob = qbs[b], kvbs[b], kebs[b], obs[b]

            @pl.loop(0, B)
            def _(e):
                sacc = jnp.zeros((_LANE,), jnp.float32)
                for h in range(H // NC):
                    c0 = HD * h
                    c1 = c0 + _LANE
                    ke0 = keb[e, pl.ds(c0, _LANE)]
                    ke1 = keb[e, pl.ds(c1, _LANE)]
                    q0 = qb[e, pl.ds(c0, _LANE)]
                    q1 = qb[e, pl.ds(c1, _LANE)]
                    k0 = kvb[e, pl.ds(c0, _LANE)] + ke0
                    k1 = kvb[e, pl.ds(c1, _LANE)] + ke1
                    t = q0 * k0 + q1 * k1
                    lg = jnp.sum(t)
                    a = jnp.exp(jnp.full((_LANE,), lg, jnp.float32))
                    v0 = kvb[e, pl.ds(DH + c0, _LANE)] + ke0
                    v1 = kvb[e, pl.ds(DH + c1, _LANE)] + ke1
                    ob[e, pl.ds(c0, _LANE)] = a * v0
                    ob[e, pl.ds(c1, _LANE)] = a * v1
                    sacc = jnp.where(lane == h, a, sacc)
                ob[e, pl.ds(DH, _LANE)] = sacc

        idx_issue(0, 0)
        idx_issue(1, 1)
        idx_wait(0)
        gather_issue(0, 0)
        gather_wait(0)
        idx_wait(1)
        gather_issue(1, 1)
        dst_save(0)
        idx_issue(0, 2)
        compute(0)
        scatter_issue(0)
        gather_wait(1)
        idx_wait(0)
        gather_issue(0, 2)
        dst_save(1)
        idx_issue(1, 3)
        compute(1)
        scatter_issue(1)

        @pl.loop(2, NBLK - 2, step=2)
        def _(g0):
            for b in range(2):
                g = g0 + b
                gather_wait(b)
                idx_wait(1 - b)
                gather_issue(1 - b, g + 1)
                scatter_wait(b)
                dst_save(b)
                idx_issue(b, g + 2)
                compute(b)
                scatter_issue(b)

        gather_wait(0)
        idx_wait(1)
        gather_issue(1, NBLK - 1)
        scatter_wait(0)
        dst_save(0)
        compute(0)
        scatter_issue(0)
        gather_wait(1)
        scatter_wait(1)
        dst_save(1)
        compute(1)
        scatter_issue(1)
        scatter_wait(0)
        scatter_wait(1)

        plsc.subcore_barrier()

        for z in range(RPT // ZR):
            r = row0 + z * ZR
            pltpu.sync_copy(agg.at[pl.ds(r, ZR)],
                            out_hbm.at[cid].at[pl.ds(r, ZR)])

    return k(qh, kvh, keh, eidx3)


NB = 2000
EB = 8000


def _ke_body(ea_ref, we_ref, be_ref, out_ref):
    ke = jnp.dot(ea_ref[...], we_ref[0],
                 preferred_element_type=jnp.float32) + be_ref[0]
    out_ref[0, 0] = ke[:, :DH]
    out_ref[1, 0] = ke[:, DH:]


def _tc_ke(eattr, We, be):
    out = pl.pallas_call(
        _ke_body,
        grid=(L, E // EB),
        in_specs=[
            pl.BlockSpec((EB, DE), lambda l, i: (i, 0)),
            pl.BlockSpec((1, DE, D), lambda l, i: (l, 0, 0)),
            pl.BlockSpec((1, 1, D), lambda l, i: (l, 0, 0)),
        ],
        out_specs=pl.BlockSpec((NC, 1, EB, DH), lambda l, i: (0, l, i, 0)),
        out_shape=jax.ShapeDtypeStruct((NC, L, E, DH), jnp.float32),
    )(eattr, We, be.reshape(L, 1, D))
    return out.reshape(NC * L, E, DH)


_INV_SQRT_HD = 1.0 / (HD ** 0.5)


def _split_qkv(qf, kf, vf, q_ref, kv_ref):
    q_ref[0] = qf[:, :DH]
    q_ref[1] = qf[:, DH:]
    kv_ref[0, :, :DH] = kf[:, :DH]
    kv_ref[0, :, DH:] = vf[:, :DH]
    kv_ref[1, :, :DH] = kf[:, DH:]
    kv_ref[1, :, DH:] = vf[:, DH:]


def _qkv_body(h_ref, wq_ref, bq_ref, wk_ref, bk_ref, wv_ref, bv_ref,
              q_ref, kv_ref):
    h = h_ref[...]
    qf = (jnp.dot(h, wq_ref[...], preferred_element_type=jnp.float32)
          + bq_ref[...]) * _INV_SQRT_HD
    kf = jnp.dot(h, wk_ref[...], preferred_element_type=jnp.float32) + bk_ref[...]
    vf = jnp.dot(h, wv_ref[...], preferred_element_type=jnp.float32) + bv_ref[...]
    _split_qkv(qf, kf, vf, q_ref, kv_ref)


def _tc_qkv0(x, wq, bq, wk, bk, wv, bv):
    wspec = pl.BlockSpec((D, D), lambda i: (0, 0))
    bspec = pl.BlockSpec((1, D), lambda i: (0, 0))
    return pl.pallas_call(
        _qkv_body,
        grid=(N // NB,),
        in_specs=[pl.BlockSpec((NB, D), lambda i: (i, 0)),
                  wspec, bspec, wspec, bspec, wspec, bspec],
        out_specs=[pl.BlockSpec((NC, NB, DH), lambda i: (0, i, 0)),
                   pl.BlockSpec((NC, NB, 2 * DH), lambda i: (0, i, 0))],
        out_shape=[jax.ShapeDtypeStruct((NC, N, DH), jnp.float32),
                   jax.ShapeDtypeStruct((NC, N, 2 * DH), jnp.float32)],
    )(x, wq.reshape(D, D), bq.reshape(1, D), wk.reshape(D, D),
      bk.reshape(1, D), wv.reshape(D, D), bv.reshape(1, D))


def _agg_norm(parts):
    agg = jnp.concatenate([parts[0, :, :DH], parts[1, :, :DH]], axis=1)
    s = jnp.concatenate([parts[0, :, DH:DH + 2],
                         parts[1, :, DH:DH + 2]], axis=1) + 1e-16
    rr = lax.broadcasted_iota(jnp.int32, (H, D), 0)
    cc = lax.broadcasted_iota(jnp.int32, (H, D), 1)
    sel = jnp.where(cc // HD == rr, 1.0, 0.0).astype(jnp.float32)
    dv = jnp.dot(s, sel, preferred_element_type=jnp.float32)
    return agg / dv


def _upd_body(p_ref, h_ref, ws_ref, bs_ref, wq_ref, bq_ref, wk_ref, bk_ref,
              wv_ref, bv_ref, h_out, q_ref, kv_ref):
    res = _agg_norm(p_ref[...])
    hn = res + jnp.dot(h_ref[...], ws_ref[...],
                       preferred_element_type=jnp.float32) + bs_ref[...]
    hn = jnp.maximum(hn, 0.0)
    h_out[...] = hn
    qf = (jnp.dot(hn, wq_ref[...], preferred_element_type=jnp.float32)
          + bq_ref[...]) * _INV_SQRT_HD
    kf = jnp.dot(hn, wk_ref[...], preferred_element_type=jnp.float32) + bk_ref[...]
    vf = jnp.dot(hn, wv_ref[...], preferred_element_type=jnp.float32) + bv_ref[...]
    _split_qkv(qf, kf, vf, q_ref, kv_ref)


def _tc_update(parts, h, ws, bs, wq, bq, wk, bk, wv, bv):
    wspec = pl.BlockSpec((D, D), lambda i: (0, 0))
    bspec = pl.BlockSpec((1, D), lambda i: (0, 0))
    return pl.pallas_call(
        _upd_body,
        grid=(N // NB,),
        in_specs=[pl.BlockSpec((NC, NB, AW), lambda i: (0, i, 0)),
                  pl.BlockSpec((NB, D), lambda i: (i, 0)),
                  wspec, bspec, wspec, bspec, wspec, bspec, wspec, bspec],
        out_specs=[pl.BlockSpec((NB, D), lambda i: (i, 0)),
                   pl.BlockSpec((NC, NB, DH), lambda i: (0, i, 0)),
                   pl.BlockSpec((NC, NB, 2 * DH), lambda i: (0, i, 0))],
        out_shape=[jax.ShapeDtypeStruct((N, D), jnp.float32),
                   jax.ShapeDtypeStruct((NC, N, DH), jnp.float32),
                   jax.ShapeDtypeStruct((NC, N, 2 * DH), jnp.float32)],
    )(parts, h, ws.reshape(D, D), bs.reshape(1, D), wq.reshape(D, D),
      bq.reshape(1, D), wk.reshape(D, D), bk.reshape(1, D),
      wv.reshape(D, D), bv.reshape(1, D))


def _last_body(p_ref, h_ref, ws_ref, bs_ref, h_out):
    res = _agg_norm(p_ref[...])
    h_out[...] = res + jnp.dot(h_ref[...], ws_ref[...],
                               preferred_element_type=jnp.float32) + bs_ref[...]


def _tc_last(parts, h, ws, bs):
    return pl.pallas_call(
        _last_body,
        grid=(N // NB,),
        in_specs=[pl.BlockSpec((NC, NB, AW), lambda i: (0, i, 0)),
                  pl.BlockSpec((NB, D), lambda i: (i, 0)),
                  pl.BlockSpec((D, D), lambda i: (0, 0)),
                  pl.BlockSpec((1, D), lambda i: (0, 0))],
        out_specs=pl.BlockSpec((NB, D), lambda i: (i, 0)),
        out_shape=jax.ShapeDtypeStruct((N, D), jnp.float32),
    )(parts, h, ws.reshape(D, D), bs.reshape(1, D))


NP = 10240
NB2 = 2048


def _embed(pooled, gamma, beta, wemb, bemb):
    mu = jnp.mean(pooled, axis=-1, keepdims=True)
    var = jnp.mean((pooled - mu) ** 2, axis=-1, keepdims=True)
    z = (pooled - mu) * lax.rsqrt(var + 1e-5) * gamma + beta
    return jnp.maximum(
        jnp.dot(z, wemb, preferred_element_type=jnp.float32) + bemb, 0.0)


def _final_body(hi_ref, bi_ref, hj_ref, bj_ref, g_ref, b_ref, we_ref, be_ref,
                out_ref, pi_acc, pj_acc):
    i = pl.program_id(0)

    @pl.when(i == 0)
    def _():
        pi_acc[...] = jnp.zeros((G, D), jnp.float32)
        pj_acc[...] = jnp.zeros((G, D), jnp.float32)

    rows = lax.broadcasted_iota(jnp.int32, (G, NB2), 0)
    mi = jnp.where(rows == bi_ref[0, 0][None, :], 1.0, 0.0).astype(jnp.float32)
    mj = jnp.where(rows == bj_ref[0, 0][None, :], 1.0, 0.0).astype(jnp.float32)
    pi_acc[...] += jnp.dot(mi, hi_ref[...], preferred_element_type=jnp.float32)
    pj_acc[...] += jnp.dot(mj, hj_ref[...], preferred_element_type=jnp.float32)

    @pl.when(i == NP // NB2 - 1)
    def _():
        ei = _embed(pi_acc[...], g_ref[...], b_ref[...], we_ref[...], be_ref[...])
        ej = _embed(pj_acc[...], g_ref[...], b_ref[...], we_ref[...], be_ref[...])
        num = jnp.sum(ei * ej, axis=-1, keepdims=True)
        na = jnp.maximum(jnp.sqrt(jnp.sum(ei * ei, axis=-1, keepdims=True)), 1e-8)
        nb = jnp.maximum(jnp.sqrt(jnp.sum(ej * ej, axis=-1, keepdims=True)), 1e-8)
        out_ref[...] = jnp.broadcast_to(num / (na * nb), (G, D))


def _tc_final(hi, bi, hj, bj, gamma, beta, wemb, bemb):
    pad = ((0, NP - N), (0, 0))
    hi = jnp.pad(hi, pad)
    hj = jnp.pad(hj, pad)
    bir = jnp.pad(bi, (0, NP - N), constant_values=G).reshape(NP // NB2, 1, NB2)
    bjr = jnp.pad(bj, (0, NP - N), constant_values=G).reshape(NP // NB2, 1, NB2)
    hspec = pl.BlockSpec((NB2, D), lambda i: (i, 0))
    ispec = pl.BlockSpec((1, 1, NB2), lambda i: (i, 0, 0))
    pspec = pl.BlockSpec((1, D), lambda i: (0, 0))
    wspec = pl.BlockSpec((D, D), lambda i: (0, 0))
    out = pl.pallas_call(
        _final_body,
        grid=(NP // NB2,),
        in_specs=[hspec, ispec, hspec, ispec, pspec, pspec, wspec, pspec],
        out_specs=pl.BlockSpec((G, D), lambda i: (0, 0)),
        out_shape=jax.ShapeDtypeStruct((G, D), jnp.float32),
        scratch_shapes=[pltpu.VMEM((G, D), jnp.float32),
                        pltpu.VMEM((G, D), jnp.float32)],
    )(hi, bir, hj, bjr, gamma.reshape(1, D), beta.reshape(1, D),
      wemb, bemb.reshape(1, D))
    return out[:, 0]


def kernel(x_i, edge_index_i, edge_attr_i, batch_i,
           x_j, edge_index_j, edge_attr_j, batch_j,
           Wq, bq, Wk, bk, Wv, bv, We, be, Wskip, bskip,
           ln_gamma, ln_beta, W_emb, b_emb):
    ke_i = _tc_ke(edge_attr_i, We, be)
    ke_j = _tc_ke(edge_attr_j, We, be)
    eidx_i = edge_index_i.reshape(2, E // B, B).transpose(1, 0, 2)
    eidx_j = edge_index_j.reshape(2, E // B, B).transpose(1, 0, 2)

    hi = x_i
    hj = x_j
    qi, kvi = _tc_qkv0(x_i, Wq[0], bq[0], Wk[0], bk[0], Wv[0], bv[0])
    qj, kvj = _tc_qkv0(x_j, Wq[0], bq[0], Wk[0], bk[0], Wv[0], bv[0])

    for l in range(L):
        pi = _sc_edge(qi, kvi, ke_i, l, eidx_i)
        pj = _sc_edge(qj, kvj, ke_j, l, eidx_j)
        if l < L - 1:
            hi, qi, kvi = _tc_update(pi, hi, Wskip[l], bskip[l], Wq[l + 1],
                                     bq[l + 1], Wk[l + 1], bk[l + 1],
                                     Wv[l + 1], bv[l + 1])
            hj, qj, kvj = _tc_update(pj, hj, Wskip[l], bskip[l], Wq[l + 1],
                                     bq[l + 1], Wk[l + 1], bk[l + 1],
                                     Wv[l + 1], bv[l + 1])
        else:
            hi = _tc_last(pi, hi, Wskip[l], bskip[l])
            hj = _tc_last(pj, hj, Wskip[l], bskip[l])

    return _tc_final(hi, batch_i, hj, batch_j,
                     ln_gamma, ln_beta, W_emb, b_emb)


# speedup vs baseline: 33.8725x; 1.1606x over previous
import functools

import jax
import jax.numpy as jnp
from jax import lax
from jax.experimental import pallas as pl
from jax.experimental.pallas import tpu as pltpu
from jax.experimental.pallas import tpu_sc as plsc

N = 10000
E = 320000
D = 128
H = 4
HD = D // H
L = 4
G = 64
DE = 16
DH = D // 2
AW = DH + 16

NC, NS = 2, 16
EPT = E // NS
B = 80
NBLK = EPT // B
RPT = N // NS
ZR = 125

_LANE = 16


def _sc_edge(qh, kvh, keh, lyr, eidx3):
    mesh = plsc.VectorSubcoreMesh(core_axis_name="c", subcore_axis_name="s")
    RB = NBLK

    @functools.partial(
        pl.kernel,
        out_type=jax.ShapeDtypeStruct((NC, N, AW), jnp.float32),
        mesh=mesh,
        compiler_params=pltpu.CompilerParams(use_tc_tiling_on_sc=False,
                                             needs_layout_passes=False),
        scratch_types=[
            pltpu.VMEM((2, B), jnp.int32),
            pltpu.VMEM((2, B), jnp.int32),
            pltpu.VMEM((B,), jnp.int32),
            pltpu.VMEM((B,), jnp.int32),
            pltpu.VMEM((B, DH), jnp.float32),
            pltpu.VMEM((B, DH), jnp.float32),
            pltpu.VMEM((B, 2 * DH), jnp.float32),
            pltpu.VMEM((B, 2 * DH), jnp.float32),
            pltpu.VMEM((B, DH), jnp.float32),
            pltpu.VMEM((B, DH), jnp.float32),
            pltpu.VMEM((B, AW), jnp.float32),
            pltpu.VMEM((B, AW), jnp.float32),
            pltpu.VMEM((ZR, AW), jnp.float32),
            pltpu.VMEM_SHARED((N, AW), jnp.float32),
            pltpu.SemaphoreType.DMA,
            pltpu.SemaphoreType.DMA,
            pltpu.SemaphoreType.DMA,
            pltpu.SemaphoreType.DMA,
            pltpu.SemaphoreType.DMA,
            pltpu.SemaphoreType.DMA,
            pltpu.SemaphoreType.DMA,
            pltpu.SemaphoreType.DMA,
            pltpu.SemaphoreType.DMA,
            pltpu.SemaphoreType.DMA,
        ],
    )
    def k(qh_hbm, kvh_hbm, keh_hbm, eidx_hbm, out_hbm,
          ix0, ix1, sd0, sd1, qb0, qb1, kvb0, kvb1, keb0, keb1, ob0, ob1,
          zb, agg, si0, si1, sq0, sq1, skv0, skv1, ske0, ske1, ss0, ss1):
        cid = lax.axis_index("c")
        sid = lax.axis_index("s")
        ixs, sds = (ix0, ix1), (sd0, sd1)
        qbs, kvbs, kebs, obs = (qb0, qb1), (kvb0, kvb1), (keb0, keb1), (ob0, ob1)
        sis, sqs = (si0, si1), (sq0, sq1)
        skvs, skes, sss = (skv0, skv1), (ske0, ske1), (ss0, ss1)

        @pl.loop(0, ZR)
        def _(r):
            for c in range(AW // _LANE):
                zb[r, pl.ds(c * _LANE, _LANE)] = jnp.zeros((_LANE,), jnp.float32)

        row0 = sid * RPT
        for z in range(RPT // ZR):
            pltpu.sync_copy(zb, agg.at[pl.ds(row0 + z * ZR, ZR)])
        plsc.subcore_barrier()

        lane = lax.iota(jnp.int32, _LANE)
        kslab = cid * L + lyr

        def idx_issue(b, g):
            pltpu.async_copy(eidx_hbm.at[sid * RB + g], ixs[b], sis[b])

        def idx_wait(b):
            pltpu.make_async_copy(eidx_hbm.at[0], ixs[b], sis[b]).wait()

        def gather_issue(b, g):
            pltpu.async_copy(qh_hbm.at[cid].at[ixs[b].at[1]], qbs[b], sqs[b])
            pltpu.async_copy(kvh_hbm.at[cid].at[ixs[b].at[0]], kvbs[b], skvs[b])
            pltpu.async_copy(keh_hbm.at[kslab].at[pl.ds((sid * RB + g) * B, B)],
                             kebs[b], skes[b])

        def gather_wait(b):
            pltpu.make_async_copy(qh_hbm.at[cid].at[ixs[b].at[1]],
                                  qbs[b], sqs[b]).wait()
            pltpu.make_async_copy(kvh_hbm.at[cid].at[ixs[b].at[0]],
                                  kvbs[b], skvs[b]).wait()
            pltpu.make_async_copy(keh_hbm.at[kslab].at[pl.ds(0, B)],
                                  kebs[b], skes[b]).wait()

        def dst_save(b):
            for c in range(B // _LANE):
                sds[b][pl.ds(c * _LANE, _LANE)] = ixs[b][1, pl.ds(c * _LANE, _LANE)]

        def scatter_issue(b):
            pltpu.async_copy(obs[b], agg.at[sds[b]], sss[b], add=True)

        def scatter_wait(b):
            pltpu.make_async_copy(obs[b], agg.at[sds[b]], sss[b]).wait()

        lane0 = jnp.zeros((_LANE,), jnp.int32)
        lane8 = jnp.full((_LANE,), 8, jnp.int32)

        def xl(x, idx):
            return x.at[idx].get(mode='promise_in_bounds')

        def compute(b):
            qb, kvb, keb, ob = qbs[b], kvbs[b], kebs[b], obs[b]

            @pl.loop(0, B)
            def _(e):
                ts = []
                for h in range(H // NC):
                    c0 = HD * h
                    c1 = c0 + _LANE
                    q0 = qb[e, pl.ds(c0, _LANE)]
                    q1 = qb[e, pl.ds(c1, _LANE)]
                    k0 = kvb[e, pl.ds(c0, _LANE)] + keb[e, pl.ds(c0, _LANE)]
                    k1 = kvb[e, pl.ds(c1, _LANE)] + keb[e, pl.ds(c1, _LANE)]
                    t = q0 * k0 + q1 * k1
                    ts.append(t + xl(t, lane ^ 8))
                m = jnp.where(lane < 8, ts[0], ts[1])
                for sh in (4, 2, 1):
                    m = m + xl(m, lane ^ sh)
                am = jnp.exp(m)
                a0 = xl(am, lane0)
                a1 = xl(am, lane8)
                for h, a in ((0, a0), (1, a1)):
                    c0 = HD * h
                    c1 = c0 + _LANE
                    v0 = kvb[e, pl.ds(DH + c0, _LANE)] + keb[e, pl.ds(c0, _LANE)]
                    v1 = kvb[e, pl.ds(DH + c1, _LANE)] + keb[e, pl.ds(c1, _LANE)]
                    ob[e, pl.ds(c0, _LANE)] = a * v0
                    ob[e, pl.ds(c1, _LANE)] = a * v1
                ob[e, pl.ds(DH, _LANE)] = jnp.where(lane == 0, a0, a1)

        idx_issue(0, 0)
        idx_issue(1, 1)
        idx_wait(0)
        gather_issue(0, 0)
        gather_wait(0)
        idx_wait(1)
        gather_issue(1, 1)
        dst_save(0)
        idx_issue(0, 2)
        compute(0)
        scatter_issue(0)
        gather_wait(1)
        idx_wait(0)
        gather_issue(0, 2)
        dst_save(1)
        idx_issue(1, 3)
        compute(1)
        scatter_issue(1)

        @pl.loop(2, NBLK - 2, step=2)
        def _(g0):
            for b in range(2):
                g = g0 + b
                gather_wait(b)
                idx_wait(1 - b)
                gather_issue(1 - b, g + 1)
                scatter_wait(b)
                dst_save(b)
                idx_issue(b, g + 2)
                compute(b)
                scatter_issue(b)

        gather_wait(0)
        idx_wait(1)
        gather_issue(1, NBLK - 1)
        scatter_wait(0)
        dst_save(0)
        compute(0)
        scatter_issue(0)
        gather_wait(1)
        scatter_wait(1)
        dst_save(1)
        compute(1)
        scatter_issue(1)
        scatter_wait(0)
        scatter_wait(1)

        plsc.subcore_barrier()

        for z in range(RPT // ZR):
            r = row0 + z * ZR
            pltpu.sync_copy(agg.at[pl.ds(r, ZR)],
                            out_hbm.at[cid].at[pl.ds(r, ZR)])

    return k(qh, kvh, keh, eidx3)


NB = 2000
EB = 8000


def _ke_body(ea_ref, we_ref, be_ref, out_ref):
    ke = jnp.dot(ea_ref[...], we_ref[0],
                 preferred_element_type=jnp.float32) + be_ref[0]
    out_ref[0, 0] = ke[:, :DH]
    out_ref[1, 0] = ke[:, DH:]


def _tc_ke(eattr, We, be):
    out = pl.pallas_call(
        _ke_body,
        grid=(L, E // EB),
        in_specs=[
            pl.BlockSpec((EB, DE), lambda l, i: (i, 0)),
            pl.BlockSpec((1, DE, D), lambda l, i: (l, 0, 0)),
            pl.BlockSpec((1, 1, D), lambda l, i: (l, 0, 0)),
        ],
        out_specs=pl.BlockSpec((NC, 1, EB, DH), lambda l, i: (0, l, i, 0)),
        out_shape=jax.ShapeDtypeStruct((NC, L, E, DH), jnp.float32),
    )(eattr, We, be.reshape(L, 1, D))
    return out.reshape(NC * L, E, DH)


_INV_SQRT_HD = 1.0 / (HD ** 0.5)


def _split_qkv(qf, kf, vf, q_ref, kv_ref):
    q_ref[0] = qf[:, :DH]
    q_ref[1] = qf[:, DH:]
    kv_ref[0, :, :DH] = kf[:, :DH]
    kv_ref[0, :, DH:] = vf[:, :DH]
    kv_ref[1, :, :DH] = kf[:, DH:]
    kv_ref[1, :, DH:] = vf[:, DH:]


def _qkv_body(h_ref, wq_ref, bq_ref, wk_ref, bk_ref, wv_ref, bv_ref,
              q_ref, kv_ref):
    h = h_ref[...]
    qf = (jnp.dot(h, wq_ref[...], preferred_element_type=jnp.float32)
          + bq_ref[...]) * _INV_SQRT_HD
    kf = jnp.dot(h, wk_ref[...], preferred_element_type=jnp.float32) + bk_ref[...]
    vf = jnp.dot(h, wv_ref[...], preferred_element_type=jnp.float32) + bv_ref[...]
    _split_qkv(qf, kf, vf, q_ref, kv_ref)


def _tc_qkv0(x, wq, bq, wk, bk, wv, bv):
    wspec = pl.BlockSpec((D, D), lambda i: (0, 0))
    bspec = pl.BlockSpec((1, D), lambda i: (0, 0))
    return pl.pallas_call(
        _qkv_body,
        grid=(N // NB,),
        in_specs=[pl.BlockSpec((NB, D), lambda i: (i, 0)),
                  wspec, bspec, wspec, bspec, wspec, bspec],
        out_specs=[pl.BlockSpec((NC, NB, DH), lambda i: (0, i, 0)),
                   pl.BlockSpec((NC, NB, 2 * DH), lambda i: (0, i, 0))],
        out_shape=[jax.ShapeDtypeStruct((NC, N, DH), jnp.float32),
                   jax.ShapeDtypeStruct((NC, N, 2 * DH), jnp.float32)],
    )(x, wq.reshape(D, D), bq.reshape(1, D), wk.reshape(D, D),
      bk.reshape(1, D), wv.reshape(D, D), bv.reshape(1, D))


def _agg_norm(parts):
    agg = jnp.concatenate([parts[0, :, :DH], parts[1, :, :DH]], axis=1)
    s = jnp.concatenate([parts[0, :, DH:DH + 2],
                         parts[1, :, DH:DH + 2]], axis=1) + 1e-16
    rr = lax.broadcasted_iota(jnp.int32, (H, D), 0)
    cc = lax.broadcasted_iota(jnp.int32, (H, D), 1)
    sel = jnp.where(cc // HD == rr, 1.0, 0.0).astype(jnp.float32)
    dv = jnp.dot(s, sel, preferred_element_type=jnp.float32)
    return agg / dv


def _upd_body(p_ref, h_ref, ws_ref, bs_ref, wq_ref, bq_ref, wk_ref, bk_ref,
              wv_ref, bv_ref, h_out, q_ref, kv_ref):
    res = _agg_norm(p_ref[...])
    hn = res + jnp.dot(h_ref[...], ws_ref[...],
                       preferred_element_type=jnp.float32) + bs_ref[...]
    hn = jnp.maximum(hn, 0.0)
    h_out[...] = hn
    qf = (jnp.dot(hn, wq_ref[...], preferred_element_type=jnp.float32)
          + bq_ref[...]) * _INV_SQRT_HD
    kf = jnp.dot(hn, wk_ref[...], preferred_element_type=jnp.float32) + bk_ref[...]
    vf = jnp.dot(hn, wv_ref[...], preferred_element_type=jnp.float32) + bv_ref[...]
    _split_qkv(qf, kf, vf, q_ref, kv_ref)


def _tc_update(parts, h, ws, bs, wq, bq, wk, bk, wv, bv):
    wspec = pl.BlockSpec((D, D), lambda i: (0, 0))
    bspec = pl.BlockSpec((1, D), lambda i: (0, 0))
    return pl.pallas_call(
        _upd_body,
        grid=(N // NB,),
        in_specs=[pl.BlockSpec((NC, NB, AW), lambda i: (0, i, 0)),
                  pl.BlockSpec((NB, D), lambda i: (i, 0)),
                  wspec, bspec, wspec, bspec, wspec, bspec, wspec, bspec],
        out_specs=[pl.BlockSpec((NB, D), lambda i: (i, 0)),
                   pl.BlockSpec((NC, NB, DH), lambda i: (0, i, 0)),
                   pl.BlockSpec((NC, NB, 2 * DH), lambda i: (0, i, 0))],
        out_shape=[jax.ShapeDtypeStruct((N, D), jnp.float32),
                   jax.ShapeDtypeStruct((NC, N, DH), jnp.float32),
                   jax.ShapeDtypeStruct((NC, N, 2 * DH), jnp.float32)],
    )(parts, h, ws.reshape(D, D), bs.reshape(1, D), wq.reshape(D, D),
      bq.reshape(1, D), wk.reshape(D, D), bk.reshape(1, D),
      wv.reshape(D, D), bv.reshape(1, D))


def _last_body(p_ref, h_ref, ws_ref, bs_ref, h_out):
    res = _agg_norm(p_ref[...])
    h_out[...] = res + jnp.dot(h_ref[...], ws_ref[...],
                               preferred_element_type=jnp.float32) + bs_ref[...]


def _tc_last(parts, h, ws, bs):
    return pl.pallas_call(
        _last_body,
        grid=(N // NB,),
        in_specs=[pl.BlockSpec((NC, NB, AW), lambda i: (0, i, 0)),
                  pl.BlockSpec((NB, D), lambda i: (i, 0)),
                  pl.BlockSpec((D, D), lambda i: (0, 0)),
                  pl.BlockSpec((1, D), lambda i: (0, 0))],
        out_specs=pl.BlockSpec((NB, D), lambda i: (i, 0)),
        out_shape=jax.ShapeDtypeStruct((N, D), jnp.float32),
    )(parts, h, ws.reshape(D, D), bs.reshape(1, D))


NP = 10240
NB2 = 2048


def _embed(pooled, gamma, beta, wemb, bemb):
    mu = jnp.mean(pooled, axis=-1, keepdims=True)
    var = jnp.mean((pooled - mu) ** 2, axis=-1, keepdims=True)
    z = (pooled - mu) * lax.rsqrt(var + 1e-5) * gamma + beta
    return jnp.maximum(
        jnp.dot(z, wemb, preferred_element_type=jnp.float32) + bemb, 0.0)


def _final_body(hi_ref, bi_ref, hj_ref, bj_ref, g_ref, b_ref, we_ref, be_ref,
                out_ref, pi_acc, pj_acc):
    i = pl.program_id(0)

    @pl.when(i == 0)
    def _():
        pi_acc[...] = jnp.zeros((G, D), jnp.float32)
        pj_acc[...] = jnp.zeros((G, D), jnp.float32)

    rows = lax.broadcasted_iota(jnp.int32, (G, NB2), 0)
    mi = jnp.where(rows == bi_ref[0, 0][None, :], 1.0, 0.0).astype(jnp.float32)
    mj = jnp.where(rows == bj_ref[0, 0][None, :], 1.0, 0.0).astype(jnp.float32)
    pi_acc[...] += jnp.dot(mi, hi_ref[...], preferred_element_type=jnp.float32)
    pj_acc[...] += jnp.dot(mj, hj_ref[...], preferred_element_type=jnp.float32)

    @pl.when(i == NP // NB2 - 1)
    def _():
        ei = _embed(pi_acc[...], g_ref[...], b_ref[...], we_ref[...], be_ref[...])
        ej = _embed(pj_acc[...], g_ref[...], b_ref[...], we_ref[...], be_ref[...])
        num = jnp.sum(ei * ej, axis=-1, keepdims=True)
        na = jnp.maximum(jnp.sqrt(jnp.sum(ei * ei, axis=-1, keepdims=True)), 1e-8)
        nb = jnp.maximum(jnp.sqrt(jnp.sum(ej * ej, axis=-1, keepdims=True)), 1e-8)
        out_ref[...] = jnp.broadcast_to(num / (na * nb), (G, D))


def _tc_final(hi, bi, hj, bj, gamma, beta, wemb, bemb):
    pad = ((0, NP - N), (0, 0))
    hi = jnp.pad(hi, pad)
    hj = jnp.pad(hj, pad)
    bir = jnp.pad(bi, (0, NP - N), constant_values=G).reshape(NP // NB2, 1, NB2)
    bjr = jnp.pad(bj, (0, NP - N), constant_values=G).reshape(NP // NB2, 1, NB2)
    hspec = pl.BlockSpec((NB2, D), lambda i: (i, 0))
    ispec = pl.BlockSpec((1, 1, NB2), lambda i: (i, 0, 0))
    pspec = pl.BlockSpec((1, D), lambda i: (0, 0))
    wspec = pl.BlockSpec((D, D), lambda i: (0, 0))
    out = pl.pallas_call(
        _final_body,
        grid=(NP // NB2,),
        in_specs=[hspec, ispec, hspec, ispec, pspec, pspec, wspec, pspec],
        out_specs=pl.BlockSpec((G, D), lambda i: (0, 0)),
        out_shape=jax.ShapeDtypeStruct((G, D), jnp.float32),
        scratch_shapes=[pltpu.VMEM((G, D), jnp.float32),
                        pltpu.VMEM((G, D), jnp.float32)],
    )(hi, bir, hj, bjr, gamma.reshape(1, D), beta.reshape(1, D),
      wemb, bemb.reshape(1, D))
    return out[:, 0]


def kernel(x_i, edge_index_i, edge_attr_i, batch_i,
           x_j, edge_index_j, edge_attr_j, batch_j,
           Wq, bq, Wk, bk, Wv, bv, We, be, Wskip, bskip,
           ln_gamma, ln_beta, W_emb, b_emb):
    ke_i = _tc_ke(edge_attr_i, We, be)
    ke_j = _tc_ke(edge_attr_j, We, be)
    eidx_i = edge_index_i.reshape(2, E // B, B).transpose(1, 0, 2)
    eidx_j = edge_index_j.reshape(2, E // B, B).transpose(1, 0, 2)

    hi = x_i
    hj = x_j
    qi, kvi = _tc_qkv0(x_i, Wq[0], bq[0], Wk[0], bk[0], Wv[0], bv[0])
    qj, kvj = _tc_qkv0(x_j, Wq[0], bq[0], Wk[0], bk[0], Wv[0], bv[0])

    for l in range(L):
        pi = _sc_edge(qi, kvi, ke_i, l, eidx_i)
        pj = _sc_edge(qj, kvj, ke_j, l, eidx_j)
        if l < L - 1:
            hi, qi, kvi = _tc_update(pi, hi, Wskip[l], bskip[l], Wq[l + 1],
                                     bq[l + 1], Wk[l + 1], bk[l + 1],
                                     Wv[l + 1], bv[l + 1])
            hj, qj, kvj = _tc_update(pj, hj, Wskip[l], bskip[l], Wq[l + 1],
                                     bq[l + 1], Wk[l + 1], bk[l + 1],
                                     Wv[l + 1], bv[l + 1])
        else:
            hi = _tc_last(pi, hi, Wskip[l], bskip[l])
            hj = _tc_last(pj, hj, Wskip[l], bskip[l])

    return _tc_final(hi, batch_i, hj, batch_j,
                     ln_gamma, ln_beta, W_emb, b_emb)
